# baseline (speedup 1.0000x reference)
import functools

import jax
import jax.numpy as jnp
from jax import lax
from jax.experimental import pallas as pl
from jax.experimental.pallas import tpu as pltpu
from jax.experimental.pallas import tpu_sc as plsc

N = 10000
E = 320000
D = 128
Q = 16384
ALPHA = 0.1

NC = 2
NS = 16
NW = NC * NS
EPT = E // NW
CHUNK = 80
NQ = EPT // CHUNK
NBUF = 4
RPT = 640
RPT_LAST = N - RPT * (NS - 1)
N_PAD = RPT * NS

_MESH = plsc.VectorSubcoreMesh(
    core_axis_name="c", subcore_axis_name="s", num_cores=NC, num_subcores=NS)


@functools.partial(
    pl.kernel,
    out_type=jax.ShapeDtypeStruct((NC * N_PAD,), jnp.float32),
    mesh=_MESH,
    scratch_types=[
        pltpu.VMEM((NQ, CHUNK), jnp.int32),
        pltpu.VMEM((CHUNK,), jnp.float32),
        pltpu.VMEM((RPT,), jnp.float32),
        pltpu.VMEM_SHARED((N_PAD,), jnp.float32),
    ],
)
def _sc_degree(col3_hbm, out_hbm, cbuf_v, ones_v, zero_v, hist_sh):
    c = lax.axis_index("c")
    s = lax.axis_index("s")
    wid = s * NC + c
    for i in range(CHUNK // 16):
        ones_v[pl.ds(i * 16, 16)] = jnp.ones((16,), jnp.float32)
    for i in range(RPT // 16):
        zero_v[pl.ds(i * 16, 16)] = jnp.zeros((16,), jnp.float32)
    pltpu.sync_copy(zero_v, hist_sh.at[pl.ds(s * RPT, RPT)])
    pltpu.sync_copy(col3_hbm.at[wid], cbuf_v)
    plsc.subcore_barrier()

    def body(q, carry):
        pltpu.sync_copy(ones_v, hist_sh.at[cbuf_v.at[q]], add=True)
        return carry

    lax.fori_loop(0, NQ, body, 0)
    plsc.subcore_barrier()
    pltpu.sync_copy(hist_sh.at[pl.ds(s * RPT, RPT)],
                    out_hbm.at[pl.ds(c * N_PAD + s * RPT, RPT)])


M8 = 2 * NBUF


@functools.partial(
    pl.kernel,
    out_type=jax.ShapeDtypeStruct((NC * N, D), jnp.float32),
    mesh=_MESH,
    scratch_types=[
        pltpu.VMEM((M8, CHUNK), jnp.int32),
        pltpu.VMEM((M8, CHUNK), jnp.int32),
        pltpu.VMEM((NBUF, CHUNK, D), jnp.float32),
        pltpu.VMEM_SHARED((N_PAD, D), jnp.float32),
    ] + [pltpu.SemaphoreType.DMA] * (2 * M8 + 2 * NBUF),
)
def _sc_scatter(hs_hbm, ei1_hbm, out_hbm,
                ridx_v, cidx_v, rows_v, agg_sh, *sems):
    sem_r = sems[0:M8]
    sem_c = sems[M8:2 * M8]
    sem_g = sems[2 * M8:2 * M8 + NBUF]
    sem_s = sems[2 * M8 + NBUF:2 * M8 + 2 * NBUF]
    c = lax.axis_index("c")
    s = lax.axis_index("s")
    wid = s * NC + c
    r0 = s * RPT
    e0 = wid * EPT
    @pl.when(s < NS - 1)
    def _():
        pltpu.sync_copy(hs_hbm.at[pl.ds(r0, RPT)], agg_sh.at[pl.ds(r0, RPT)])

    @pl.when(s == NS - 1)
    def _():
        pltpu.sync_copy(hs_hbm.at[pl.ds((NS - 1) * RPT, RPT_LAST)],
                        agg_sh.at[pl.ds((NS - 1) * RPT, RPT_LAST)])

    plsc.subcore_barrier()

    def idx_load(q, b8):
        pltpu.async_copy(ei1_hbm.at[pl.ds(e0 + q * CHUNK, CHUNK)],
                         ridx_v.at[b8], sem_r[b8])
        pltpu.async_copy(ei1_hbm.at[pl.ds(E + e0 + q * CHUNK, CHUNK)],
                         cidx_v.at[b8], sem_c[b8])

    def idx_wait_row(q, b8):
        pltpu.make_async_copy(ei1_hbm.at[pl.ds(e0 + q * CHUNK, CHUNK)],
                              ridx_v.at[b8], sem_r[b8]).wait()

    def idx_wait_col(q, b8):
        pltpu.make_async_copy(ei1_hbm.at[pl.ds(E + e0 + q * CHUNK, CHUNK)],
                              cidx_v.at[b8], sem_c[b8]).wait()

    def gather(b8, b4):
        pltpu.async_copy(hs_hbm.at[ridx_v.at[b8]], rows_v.at[b4], sem_g[b4])

    def gather_wait(b8, b4):
        pltpu.make_async_copy(hs_hbm.at[ridx_v.at[b8]],
                              rows_v.at[b4], sem_g[b4]).wait()

    def scat(b8, b4):
        pltpu.async_copy(rows_v.at[b4], agg_sh.at[cidx_v.at[b8]],
                         sem_s[b4], add=True)

    def scat_wait(b8, b4):
        pltpu.make_async_copy(rows_v.at[b4], agg_sh.at[cidx_v.at[b8]],
                              sem_s[b4]).wait()

    for b in range(NBUF - 1):
        idx_load(b, b)
    idx_wait_row(0, 0)
    gather(0, 0)

    def body(q, carry):
        par = lax.rem(q, M8)
        for b8 in range(M8):
            b4 = b8 % NBUF
            n8 = (b8 + 1) % M8
            n4 = (b8 + 1) % NBUF
            p8 = (b8 + NBUF - 1) % M8

            @pl.when((par == b8) & (q + NBUF - 1 < NQ))
            def _(p8=p8):
                idx_load(q + NBUF - 1, p8)

            @pl.when((par == b8) & (q + 1 < NQ))
            def _(n8=n8, n4=n4):
                @pl.when(q + 1 >= NBUF)
                def _():
                    scat_wait((n8 + NBUF) % M8, n4)
                idx_wait_row(q + 1, n8)
                gather(n8, n4)

            @pl.when(par == b8)
            def _(b8=b8, b4=b4):
                gather_wait(b8, b4)
                idx_wait_col(q, b8)
                scat(b8, b4)
        return carry

    lax.fori_loop(0, NQ, body, 0)
    for qq in range(NQ - NBUF, NQ):
        scat_wait(qq % M8, qq % NBUF)
    plsc.subcore_barrier()

    @pl.when(s < NS - 1)
    def _():
        pltpu.sync_copy(agg_sh.at[pl.ds(r0, RPT)],
                        out_hbm.at[pl.ds(c * N + r0, RPT)])

    @pl.when(s == NS - 1)
    def _():
        pltpu.sync_copy(agg_sh.at[pl.ds((NS - 1) * RPT, RPT_LAST)],
                        out_hbm.at[pl.ds(c * N + (NS - 1) * RPT, RPT_LAST)])


QPT = 2 * Q // NW
GCHUNK = 128


@functools.partial(
    pl.kernel,
    out_type=jax.ShapeDtypeStruct((2 * Q, D), jnp.float32),
    mesh=_MESH,
    scratch_types=[
        pltpu.VMEM((QPT,), jnp.int32),
        pltpu.VMEM((2, GCHUNK, D), jnp.float32),
        pltpu.SemaphoreType.DMA,
        pltpu.SemaphoreType.DMA,
    ],
)
def _sc_link(tab_hbm, idx_hbm, out_hbm, idx_v, rows_v, sem0, sem1):
    c = lax.axis_index("c")
    s = lax.axis_index("s")
    wid = s * NC + c
    base0 = wid * QPT
    pltpu.sync_copy(idx_hbm.at[pl.ds(base0, QPT)], idx_v)
    sems = (sem0, sem1)

    def gather(q, b):
        pltpu.async_copy(tab_hbm.at[idx_v.at[pl.ds(q * GCHUNK, GCHUNK)]],
                         rows_v.at[b], sems[b])

    def drain(q, b):
        pltpu.make_async_copy(tab_hbm.at[idx_v.at[pl.ds(q * GCHUNK, GCHUNK)]],
                              rows_v.at[b], sems[b]).wait()
        pltpu.sync_copy(rows_v.at[b],
                        out_hbm.at[pl.ds(base0 + q * GCHUNK, GCHUNK)])

    gather(0, 0)

    def body(q, carry):
        for b in range(2):
            @pl.when(lax.rem(q, 2) == b)
            def _(b=b):
                @pl.when(q + 1 < QPT // GCHUNK)
                def _():
                    gather(q + 1, 1 - b)
                drain(q, b)
        return carry

    lax.fori_loop(0, QPT // GCHUNK, body, 0)


BR = 1000


def _scales(dg):
    d = dg[:, 0:1] + dg[:, 1:2] + 1.0
    dis = lax.rsqrt(d)
    return (1.0 - ALPHA) * dis, ALPHA * lax.sqrt(d) - (1.0 - ALPHA) * dis, dis


def _tc_mm1_body(x_ref, w_ref, b_ref, h_ref):
    h_ref[...] = lax.dot_general(
        x_ref[...], w_ref[...], (((1,), (1,)), ((), ())),
        preferred_element_type=jnp.float32) + b_ref[...]


def _tc_mm1(x, w1, b1r):
    return pl.pallas_call(
        _tc_mm1_body,
        grid=(N // BR,),
        in_specs=[
            pl.BlockSpec((BR, D), lambda i: (i, 0)),
            pl.BlockSpec((D, D), lambda i: (0, 0)),
            pl.BlockSpec((1, D), lambda i: (0, 0)),
        ],
        out_specs=pl.BlockSpec((BR, D), lambda i: (i, 0)),
        out_shape=jax.ShapeDtypeStruct((N, D), jnp.float32),
    )(x, w1, b1r)


def _tc_scale1_body(h_ref, dg_ref, hs_ref):
    d = dg_ref[:, 0:1] + dg_ref[:, 1:2] + 1.0
    hs_ref[...] = h_ref[...] * lax.rsqrt(d)


def _tc_scale1(h, degcols):
    return pl.pallas_call(
        _tc_scale1_body,
        grid=(N // BR,),
        in_specs=[
            pl.BlockSpec((BR, D), lambda i: (i, 0)),
            pl.BlockSpec((BR, 2), lambda i: (i, 0)),
        ],
        out_specs=pl.BlockSpec((BR, D), lambda i: (i, 0)),
        out_shape=jax.ShapeDtypeStruct((N, D), jnp.float32),
    )(h, degcols)


def _tc_stage2_body(hs_ref, s0_ref, s1_ref, w_ref, b_ref, dg_ref, h2s_ref):
    ca, cb, dis = _scales(dg_ref[...])
    out1 = jnp.maximum(ca * (s0_ref[...] + s1_ref[...]) + cb * hs_ref[...],
                       0.0)
    h2 = lax.dot_general(out1, w_ref[...], (((1,), (1,)), ((), ())),
                         preferred_element_type=jnp.float32) + b_ref[...]
    h2s_ref[...] = h2 * dis


def _tc_stage2(h1s, s2n, w2, b2r, degcols):
    return pl.pallas_call(
        _tc_stage2_body,
        grid=(N // BR,),
        in_specs=[
            pl.BlockSpec((BR, D), lambda i: (i, 0)),
            pl.BlockSpec((BR, D), lambda i: (i, 0)),
            pl.BlockSpec((BR, D), lambda i: (i + N // BR, 0)),
            pl.BlockSpec((D, D), lambda i: (0, 0)),
            pl.BlockSpec((1, D), lambda i: (0, 0)),
            pl.BlockSpec((BR, 2), lambda i: (i, 0)),
        ],
        out_specs=pl.BlockSpec((BR, D), lambda i: (i, 0)),
        out_shape=jax.ShapeDtypeStruct((N, D), jnp.float32),
    )(h1s, s2n, s2n, w2, b2r, degcols)


def _tc_stage3_body(hs_ref, s0_ref, s1_ref, dg_ref, o_ref):
    ca, cb, _ = _scales(dg_ref[...])
    o_ref[...] = jnp.maximum(
        ca * (s0_ref[...] + s1_ref[...]) + cb * hs_ref[...], 0.0)


def _tc_stage3(h2s, s2n, degcols):
    return pl.pallas_call(
        _tc_stage3_body,
        grid=(N // BR,),
        in_specs=[
            pl.BlockSpec((BR, D), lambda i: (i, 0)),
            pl.BlockSpec((BR, D), lambda i: (i, 0)),
            pl.BlockSpec((BR, D), lambda i: (i + N // BR, 0)),
            pl.BlockSpec((BR, 2), lambda i: (i, 0)),
        ],
        out_specs=pl.BlockSpec((BR, D), lambda i: (i, 0)),
        out_shape=jax.ShapeDtypeStruct((N, D), jnp.float32),
    )(h2s, s2n, s2n, degcols)


def _tc_head_body(g0_ref, g1_ref, wa_ref, wb_ref, b3_ref, o_ref):
    logits = (
        lax.dot_general(g0_ref[...], wa_ref[...], (((1,), (1,)), ((), ())),
                        preferred_element_type=jnp.float32)
        + lax.dot_general(g1_ref[...], wb_ref[...], (((1,), (1,)), ((), ())),
                          preferred_element_type=jnp.float32)
        + b3_ref[...])
    m = jnp.max(logits, axis=1, keepdims=True)
    lse = m + jnp.log(jnp.sum(jnp.exp(logits - m), axis=1, keepdims=True))
    o_ref[...] = logits - lse


BQ = 2048


def _tc_head(g, w3a, w3b, b3r):
    return pl.pallas_call(
        _tc_head_body,
        grid=(Q // BQ,),
        in_specs=[
            pl.BlockSpec((BQ, D), lambda i: (i, 0)),
            pl.BlockSpec((BQ, D), lambda i: (i + Q // BQ, 0)),
            pl.BlockSpec((2, D), lambda i: (0, 0)),
            pl.BlockSpec((2, D), lambda i: (0, 0)),
            pl.BlockSpec((1, 2), lambda i: (0, 0)),
        ],
        out_specs=pl.BlockSpec((BQ, 2), lambda i: (i, 0)),
        out_shape=jax.ShapeDtypeStruct((Q, 2), jnp.float32),
    )(g, g, w3a, w3b, b3r)


def kernel(x, edge_index, index, W1, b1, W2, b2, W3, b3):
    ei = edge_index.astype(jnp.int32)
    ei1 = ei.reshape(2 * E)
    col3 = ei[1].reshape(NW, NQ, CHUNK)
    idx_all = index.astype(jnp.int32).T.reshape(2 * Q)
    b1r = b1.reshape(1, D)
    b2r = b2.reshape(1, D)
    b3r = b3.reshape(1, 2)
    w3a = W3[:, :D]
    w3b = W3[:, D:]

    deg_flat = _sc_degree(col3)
    h1 = _tc_mm1(x, W1, b1r)
    degcols = deg_flat.reshape(NC, N_PAD).T[:N]
    h1s = _tc_scale1(h1, degcols)
    s = _sc_scatter(h1s, ei1)
    h2s = _tc_stage2(h1s, s, W2, b2r, degcols)
    s2 = _sc_scatter(h2s, ei1)
    out2 = _tc_stage3(h2s, s2, degcols)
    g = _sc_link(out2, idx_all)
    return _tc_head(g, w3a, w3b, b3r)


# speedup vs baseline: 1.0029x; 1.0029x over previous
import functools

import jax
import jax.numpy as jnp
from jax import lax
from jax.experimental import pallas as pl
from jax.experimental.pallas import tpu as pltpu
from jax.experimental.pallas import tpu_sc as plsc

N = 10000
E = 320000
D = 128
Q = 16384
ALPHA = 0.1

NC = 2
NS = 16
NW = NC * NS
EPT = E // NW
CHUNK = 80
NQ = EPT // CHUNK
NBUF = 4
RPT = 640
RPT_LAST = N - RPT * (NS - 1)
N_PAD = RPT * NS

_MESH = plsc.VectorSubcoreMesh(
    core_axis_name="c", subcore_axis_name="s", num_cores=NC, num_subcores=NS)


@functools.partial(
    pl.kernel,
    out_type=jax.ShapeDtypeStruct((NC * N_PAD,), jnp.float32),
    mesh=_MESH,
    scratch_types=[
        pltpu.VMEM((NQ, CHUNK), jnp.int32),
        pltpu.VMEM((CHUNK,), jnp.float32),
        pltpu.VMEM((RPT,), jnp.float32),
        pltpu.VMEM_SHARED((N_PAD,), jnp.float32),
    ],
)
def _sc_degree(col3_hbm, out_hbm, cbuf_v, ones_v, zero_v, hist_sh):
    c = lax.axis_index("c")
    s = lax.axis_index("s")
    wid = s * NC + c
    for i in range(CHUNK // 16):
        ones_v[pl.ds(i * 16, 16)] = jnp.ones((16,), jnp.float32)
    for i in range(RPT // 16):
        zero_v[pl.ds(i * 16, 16)] = jnp.zeros((16,), jnp.float32)
    pltpu.sync_copy(zero_v, hist_sh.at[pl.ds(s * RPT, RPT)])
    pltpu.sync_copy(col3_hbm.at[wid], cbuf_v)
    plsc.subcore_barrier()

    def body(q, carry):
        pltpu.sync_copy(ones_v, hist_sh.at[cbuf_v.at[q]], add=True)
        return carry

    lax.fori_loop(0, NQ, body, 0)
    plsc.subcore_barrier()
    pltpu.sync_copy(hist_sh.at[pl.ds(s * RPT, RPT)],
                    out_hbm.at[pl.ds(c * N_PAD + s * RPT, RPT)])


M8 = 2 * NBUF


@functools.partial(
    pl.kernel,
    out_type=jax.ShapeDtypeStruct((NC * N, D), jnp.float32),
    mesh=_MESH,
    scratch_types=[
        pltpu.VMEM((M8, CHUNK), jnp.int32),
        pltpu.VMEM((M8, CHUNK), jnp.int32),
        pltpu.VMEM((NBUF, CHUNK, D), jnp.float32),
        pltpu.VMEM_SHARED((N_PAD, D), jnp.float32),
    ] + [pltpu.SemaphoreType.DMA] * (2 * M8 + 2 * NBUF),
)
def _sc_scatter(hs_hbm, ei1_hbm, out_hbm,
                ridx_v, cidx_v, rows_v, agg_sh, *sems):
    sem_r = sems[0:M8]
    sem_c = sems[M8:2 * M8]
    sem_g = sems[2 * M8:2 * M8 + NBUF]
    sem_s = sems[2 * M8 + NBUF:2 * M8 + 2 * NBUF]
    c = lax.axis_index("c")
    s = lax.axis_index("s")
    wid = s * NC + c
    r0 = s * RPT
    e0 = wid * EPT
    @pl.when(s < NS - 1)
    def _():
        pltpu.sync_copy(hs_hbm.at[pl.ds(r0, RPT)], agg_sh.at[pl.ds(r0, RPT)])

    @pl.when(s == NS - 1)
    def _():
        pltpu.sync_copy(hs_hbm.at[pl.ds((NS - 1) * RPT, RPT_LAST)],
                        agg_sh.at[pl.ds((NS - 1) * RPT, RPT_LAST)])

    plsc.subcore_barrier()

    def idx_load(q, b8):
        pltpu.async_copy(ei1_hbm.at[pl.ds(e0 + q * CHUNK, CHUNK)],
                         ridx_v.at[b8], sem_r[b8])
        pltpu.async_copy(ei1_hbm.at[pl.ds(E + e0 + q * CHUNK, CHUNK)],
                         cidx_v.at[b8], sem_c[b8])

    def idx_wait_row(q, b8):
        pltpu.make_async_copy(ei1_hbm.at[pl.ds(e0 + q * CHUNK, CHUNK)],
                              ridx_v.at[b8], sem_r[b8]).wait()

    def idx_wait_col(q, b8):
        pltpu.make_async_copy(ei1_hbm.at[pl.ds(E + e0 + q * CHUNK, CHUNK)],
                              cidx_v.at[b8], sem_c[b8]).wait()

    def gather(b8, b4):
        pltpu.async_copy(hs_hbm.at[ridx_v.at[b8]], rows_v.at[b4], sem_g[b4])

    def gather_wait(b8, b4):
        pltpu.make_async_copy(hs_hbm.at[ridx_v.at[b8]],
                              rows_v.at[b4], sem_g[b4]).wait()

    def scat(b8, b4):
        pltpu.async_copy(rows_v.at[b4], agg_sh.at[cidx_v.at[b8]],
                         sem_s[b4], add=True)

    def scat_wait(b8, b4):
        pltpu.make_async_copy(rows_v.at[b4], agg_sh.at[cidx_v.at[b8]],
                              sem_s[b4]).wait()

    for b in range(NBUF - 1):
        idx_load(b, b)
    idx_wait_row(0, 0)
    gather(0, 0)

    def body(q, carry):
        par = lax.rem(q, M8)
        for b8 in range(M8):
            b4 = b8 % NBUF
            n8 = (b8 + 1) % M8
            n4 = (b8 + 1) % NBUF
            p8 = (b8 + NBUF - 1) % M8

            @pl.when((par == b8) & (q + NBUF - 1 < NQ))
            def _(p8=p8):
                idx_load(q + NBUF - 1, p8)

            @pl.when((par == b8) & (q + 1 < NQ))
            def _(n8=n8, n4=n4):
                @pl.when(q + 1 >= NBUF)
                def _():
                    scat_wait((n8 + NBUF) % M8, n4)
                idx_wait_row(q + 1, n8)
                gather(n8, n4)

            @pl.when(par == b8)
            def _(b8=b8, b4=b4):
                gather_wait(b8, b4)
                idx_wait_col(q, b8)
                scat(b8, b4)
        return carry

    lax.fori_loop(0, NQ, body, 0)
    for qq in range(NQ - NBUF, NQ):
        scat_wait(qq % M8, qq % NBUF)
    plsc.subcore_barrier()

    @pl.when(s < NS - 1)
    def _():
        pltpu.sync_copy(agg_sh.at[pl.ds(r0, RPT)],
                        out_hbm.at[pl.ds(c * N + r0, RPT)])

    @pl.when(s == NS - 1)
    def _():
        pltpu.sync_copy(agg_sh.at[pl.ds((NS - 1) * RPT, RPT_LAST)],
                        out_hbm.at[pl.ds(c * N + (NS - 1) * RPT, RPT_LAST)])


QPT = 2 * Q // NW
GCHUNK = 128


@functools.partial(
    pl.kernel,
    out_type=jax.ShapeDtypeStruct((2 * Q, D), jnp.float32),
    mesh=_MESH,
    scratch_types=[
        pltpu.VMEM((QPT,), jnp.int32),
        pltpu.VMEM((2, GCHUNK, D), jnp.float32),
        pltpu.SemaphoreType.DMA,
        pltpu.SemaphoreType.DMA,
    ],
)
def _sc_link(tab_hbm, idx_hbm, out_hbm, idx_v, rows_v, sem0, sem1):
    c = lax.axis_index("c")
    s = lax.axis_index("s")
    wid = s * NC + c
    base0 = wid * QPT
    pltpu.sync_copy(idx_hbm.at[pl.ds(base0, QPT)], idx_v)
    sems = (sem0, sem1)

    def gather(q, b):
        pltpu.async_copy(tab_hbm.at[idx_v.at[pl.ds(q * GCHUNK, GCHUNK)]],
                         rows_v.at[b], sems[b])

    def drain(q, b):
        pltpu.make_async_copy(tab_hbm.at[idx_v.at[pl.ds(q * GCHUNK, GCHUNK)]],
                              rows_v.at[b], sems[b]).wait()
        pltpu.sync_copy(rows_v.at[b],
                        out_hbm.at[pl.ds(base0 + q * GCHUNK, GCHUNK)])

    gather(0, 0)

    def body(q, carry):
        for b in range(2):
            @pl.when(lax.rem(q, 2) == b)
            def _(b=b):
                @pl.when(q + 1 < QPT // GCHUNK)
                def _():
                    gather(q + 1, 1 - b)
                drain(q, b)
        return carry

    lax.fori_loop(0, QPT // GCHUNK, body, 0)


BR = 1000


def _scales(dg):
    d = dg[:, 0:1] + dg[:, 1:2] + 1.0
    dis = lax.rsqrt(d)
    return (1.0 - ALPHA) * dis, ALPHA * lax.sqrt(d) - (1.0 - ALPHA) * dis, dis


def _tc_stage1_body(x_ref, w_ref, b_ref, dg_ref, hs_ref):
    h = lax.dot_general(x_ref[...], w_ref[...], (((1,), (1,)), ((), ())),
                        preferred_element_type=jnp.float32) + b_ref[...]
    d = dg_ref[:, 0:1] + dg_ref[:, 1:2] + 1.0
    hs_ref[...] = h * lax.rsqrt(d)


def _tc_stage1(x, w1, b1r, degcols):
    return pl.pallas_call(
        _tc_stage1_body,
        grid=(N // BR,),
        in_specs=[
            pl.BlockSpec((BR, D), lambda i: (i, 0)),
            pl.BlockSpec((D, D), lambda i: (0, 0)),
            pl.BlockSpec((1, D), lambda i: (0, 0)),
            pl.BlockSpec((BR, 2), lambda i: (i, 0)),
        ],
        out_specs=pl.BlockSpec((BR, D), lambda i: (i, 0)),
        out_shape=jax.ShapeDtypeStruct((N, D), jnp.float32),
    )(x, w1, b1r, degcols)


def _tc_stage2_body(hs_ref, s0_ref, s1_ref, w_ref, b_ref, dg_ref, h2s_ref):
    ca, cb, dis = _scales(dg_ref[...])
    out1 = jnp.maximum(ca * (s0_ref[...] + s1_ref[...]) + cb * hs_ref[...],
                       0.0)
    h2 = lax.dot_general(out1, w_ref[...], (((1,), (1,)), ((), ())),
                         preferred_element_type=jnp.float32) + b_ref[...]
    h2s_ref[...] = h2 * dis


def _tc_stage2(h1s, s2n, w2, b2r, degcols):
    return pl.pallas_call(
        _tc_stage2_body,
        grid=(N // BR,),
        in_specs=[
            pl.BlockSpec((BR, D), lambda i: (i, 0)),
            pl.BlockSpec((BR, D), lambda i: (i, 0)),
            pl.BlockSpec((BR, D), lambda i: (i + N // BR, 0)),
            pl.BlockSpec((D, D), lambda i: (0, 0)),
            pl.BlockSpec((1, D), lambda i: (0, 0)),
            pl.BlockSpec((BR, 2), lambda i: (i, 0)),
        ],
        out_specs=pl.BlockSpec((BR, D), lambda i: (i, 0)),
        out_shape=jax.ShapeDtypeStruct((N, D), jnp.float32),
    )(h1s, s2n, s2n, w2, b2r, degcols)


def _tc_stage3_body(hs_ref, s0_ref, s1_ref, dg_ref, o_ref):
    ca, cb, _ = _scales(dg_ref[...])
    o_ref[...] = jnp.maximum(
        ca * (s0_ref[...] + s1_ref[...]) + cb * hs_ref[...], 0.0)


def _tc_stage3(h2s, s2n, degcols):
    return pl.pallas_call(
        _tc_stage3_body,
        grid=(N // BR,),
        in_specs=[
            pl.BlockSpec((BR, D), lambda i: (i, 0)),
            pl.BlockSpec((BR, D), lambda i: (i, 0)),
            pl.BlockSpec((BR, D), lambda i: (i + N // BR, 0)),
            pl.BlockSpec((BR, 2), lambda i: (i, 0)),
        ],
        out_specs=pl.BlockSpec((BR, D), lambda i: (i, 0)),
        out_shape=jax.ShapeDtypeStruct((N, D), jnp.float32),
    )(h2s, s2n, s2n, degcols)


def _tc_head_body(g0_ref, g1_ref, wa_ref, wb_ref, b3_ref, o_ref):
    logits = (
        lax.dot_general(g0_ref[...], wa_ref[...], (((1,), (1,)), ((), ())),
                        preferred_element_type=jnp.float32)
        + lax.dot_general(g1_ref[...], wb_ref[...], (((1,), (1,)), ((), ())),
                          preferred_element_type=jnp.float32)
        + b3_ref[...])
    m = jnp.max(logits, axis=1, keepdims=True)
    lse = m + jnp.log(jnp.sum(jnp.exp(logits - m), axis=1, keepdims=True))
    o_ref[...] = logits - lse


BQ = 2048


def _tc_head(g, w3a, w3b, b3r):
    return pl.pallas_call(
        _tc_head_body,
        grid=(Q // BQ,),
        in_specs=[
            pl.BlockSpec((BQ, D), lambda i: (i, 0)),
            pl.BlockSpec((BQ, D), lambda i: (i + Q // BQ, 0)),
            pl.BlockSpec((2, D), lambda i: (0, 0)),
            pl.BlockSpec((2, D), lambda i: (0, 0)),
            pl.BlockSpec((1, 2), lambda i: (0, 0)),
        ],
        out_specs=pl.BlockSpec((BQ, 2), lambda i: (i, 0)),
        out_shape=jax.ShapeDtypeStruct((Q, 2), jnp.float32),
    )(g, g, w3a, w3b, b3r)


def kernel(x, edge_index, index, W1, b1, W2, b2, W3, b3):
    ei = edge_index.astype(jnp.int32)
    ei1 = ei.reshape(2 * E)
    col3 = ei[1].reshape(NW, NQ, CHUNK)
    idx_all = index.astype(jnp.int32).T.reshape(2 * Q)
    b1r = b1.reshape(1, D)
    b2r = b2.reshape(1, D)
    b3r = b3.reshape(1, 2)
    w3a = W3[:, :D]
    w3b = W3[:, D:]

    deg_flat = _sc_degree(col3)
    degcols = deg_flat.reshape(NC, N_PAD).T[:N]

    h1s = _tc_stage1(x, W1, b1r, degcols)
    s = _sc_scatter(h1s, ei1)
    h2s = _tc_stage2(h1s, s, W2, b2r, degcols)
    s2 = _sc_scatter(h2s, ei1)
    out2 = _tc_stage3(h2s, s2, degcols)
    g = _sc_link(out2, idx_all)
    return _tc_head(g, w3a, w3b, b3r)


# speedup vs baseline: 1.0275x; 1.0245x over previous
import functools

import jax
import jax.numpy as jnp
from jax import lax
from jax.experimental import pallas as pl
from jax.experimental.pallas import tpu as pltpu
from jax.experimental.pallas import tpu_sc as plsc

N = 10000
E = 320000
D = 128
Q = 16384
ALPHA = 0.1

NC = 2
NS = 16
NW = NC * NS
EPT = E // NW
CHUNK = 80
NQ = EPT // CHUNK
NBUF = 4
RPT = 640
RPT_LAST = N - RPT * (NS - 1)
N_PAD = RPT * NS

_MESH = plsc.VectorSubcoreMesh(
    core_axis_name="c", subcore_axis_name="s", num_cores=NC, num_subcores=NS)


@functools.partial(
    pl.kernel,
    out_type=jax.ShapeDtypeStruct((NC * N_PAD,), jnp.float32),
    mesh=_MESH,
    scratch_types=[
        pltpu.VMEM((NQ, CHUNK), jnp.int32),
        pltpu.VMEM((CHUNK,), jnp.float32),
        pltpu.VMEM((RPT,), jnp.float32),
        pltpu.VMEM_SHARED((N_PAD,), jnp.float32),
    ],
)
def _sc_degree(ei3_hbm, out_hbm, cbuf_v, ones_v, zero_v, hist_sh):
    c = lax.axis_index("c")
    s = lax.axis_index("s")
    wid = s * NC + c
    for i in range(CHUNK // 16):
        ones_v[pl.ds(i * 16, 16)] = jnp.ones((16,), jnp.float32)
    for i in range(RPT // 16):
        zero_v[pl.ds(i * 16, 16)] = jnp.zeros((16,), jnp.float32)
    pltpu.sync_copy(zero_v, hist_sh.at[pl.ds(s * RPT, RPT)])
    pltpu.sync_copy(ei3_hbm.at[NW + wid], cbuf_v)
    plsc.subcore_barrier()

    def body(q, carry):
        pltpu.sync_copy(ones_v, hist_sh.at[cbuf_v.at[q]], add=True)
        return carry

    lax.fori_loop(0, NQ, body, 0)
    plsc.subcore_barrier()
    pltpu.sync_copy(hist_sh.at[pl.ds(s * RPT, RPT)],
                    out_hbm.at[pl.ds(c * N_PAD + s * RPT, RPT)])


M8 = 2 * NBUF


@functools.partial(
    pl.kernel,
    out_type=jax.ShapeDtypeStruct((NC * N, D), jnp.float32),
    mesh=_MESH,
    scratch_types=[
        pltpu.VMEM((M8, CHUNK), jnp.int32),
        pltpu.VMEM((M8, CHUNK), jnp.int32),
        pltpu.VMEM((NBUF, CHUNK, D), jnp.float32),
        pltpu.VMEM_SHARED((N_PAD, D), jnp.float32),
    ] + [pltpu.SemaphoreType.DMA] * (2 * M8 + 2 * NBUF),
)
def _sc_scatter(hs_hbm, ei1_hbm, out_hbm,
                ridx_v, cidx_v, rows_v, agg_sh, *sems):
    sem_r = sems[0:M8]
    sem_c = sems[M8:2 * M8]
    sem_g = sems[2 * M8:2 * M8 + NBUF]
    sem_s = sems[2 * M8 + NBUF:2 * M8 + 2 * NBUF]
    c = lax.axis_index("c")
    s = lax.axis_index("s")
    wid = s * NC + c
    r0 = s * RPT
    e0 = wid * EPT
    @pl.when(s < NS - 1)
    def _():
        pltpu.sync_copy(hs_hbm.at[pl.ds(r0, RPT)], agg_sh.at[pl.ds(r0, RPT)])

    @pl.when(s == NS - 1)
    def _():
        pltpu.sync_copy(hs_hbm.at[pl.ds((NS - 1) * RPT, RPT_LAST)],
                        agg_sh.at[pl.ds((NS - 1) * RPT, RPT_LAST)])

    plsc.subcore_barrier()

    def idx_load(q, b8):
        pltpu.async_copy(ei1_hbm.at[pl.ds(e0 + q * CHUNK, CHUNK)],
                         ridx_v.at[b8], sem_r[b8])
        pltpu.async_copy(ei1_hbm.at[pl.ds(E + e0 + q * CHUNK, CHUNK)],
                         cidx_v.at[b8], sem_c[b8])

    def idx_wait_row(q, b8):
        pltpu.make_async_copy(ei1_hbm.at[pl.ds(e0 + q * CHUNK, CHUNK)],
                              ridx_v.at[b8], sem_r[b8]).wait()

    def idx_wait_col(q, b8):
        pltpu.make_async_copy(ei1_hbm.at[pl.ds(E + e0 + q * CHUNK, CHUNK)],
                              cidx_v.at[b8], sem_c[b8]).wait()

    def gather(b8, b4):
        pltpu.async_copy(hs_hbm.at[ridx_v.at[b8]], rows_v.at[b4], sem_g[b4])

    def gather_wait(b8, b4):
        pltpu.make_async_copy(hs_hbm.at[ridx_v.at[b8]],
                              rows_v.at[b4], sem_g[b4]).wait()

    def scat(b8, b4):
        pltpu.async_copy(rows_v.at[b4], agg_sh.at[cidx_v.at[b8]],
                         sem_s[b4], add=True)

    def scat_wait(b8, b4):
        pltpu.make_async_copy(rows_v.at[b4], agg_sh.at[cidx_v.at[b8]],
                              sem_s[b4]).wait()

    for b in range(NBUF - 1):
        idx_load(b, b)
    idx_wait_row(0, 0)
    gather(0, 0)

    def body(q, carry):
        par = lax.rem(q, M8)
        for b8 in range(M8):
            b4 = b8 % NBUF
            n8 = (b8 + 1) % M8
            n4 = (b8 + 1) % NBUF
            p8 = (b8 + NBUF - 1) % M8

            @pl.when((par == b8) & (q + NBUF - 1 < NQ))
            def _(p8=p8):
                idx_load(q + NBUF - 1, p8)

            @pl.when((par == b8) & (q + 1 < NQ))
            def _(n8=n8, n4=n4):
                @pl.when(q + 1 >= NBUF)
                def _():
                    scat_wait((n8 + NBUF) % M8, n4)
                idx_wait_row(q + 1, n8)
                gather(n8, n4)

            @pl.when(par == b8)
            def _(b8=b8, b4=b4):
                gather_wait(b8, b4)
                idx_wait_col(q, b8)
                scat(b8, b4)
        return carry

    lax.fori_loop(0, NQ, body, 0)
    for qq in range(NQ - NBUF, NQ):
        scat_wait(qq % M8, qq % NBUF)
    plsc.subcore_barrier()

    @pl.when(s < NS - 1)
    def _():
        pltpu.sync_copy(agg_sh.at[pl.ds(r0, RPT)],
                        out_hbm.at[pl.ds(c * N + r0, RPT)])

    @pl.when(s == NS - 1)
    def _():
        pltpu.sync_copy(agg_sh.at[pl.ds((NS - 1) * RPT, RPT_LAST)],
                        out_hbm.at[pl.ds(c * N + (NS - 1) * RPT, RPT_LAST)])


QPT = 2 * Q // NW
GCHUNK = 128


@functools.partial(
    pl.kernel,
    out_type=jax.ShapeDtypeStruct((2 * Q, D), jnp.float32),
    mesh=_MESH,
    scratch_types=[
        pltpu.VMEM((QPT,), jnp.int32),
        pltpu.VMEM((2, GCHUNK, D), jnp.float32),
        pltpu.SemaphoreType.DMA,
        pltpu.SemaphoreType.DMA,
    ],
)
def _sc_link(tab_hbm, idx_hbm, out_hbm, idx_v, rows_v, sem0, sem1):
    c = lax.axis_index("c")
    s = lax.axis_index("s")
    wid = s * NC + c
    base0 = wid * QPT
    pltpu.sync_copy(idx_hbm.at[pl.ds(base0, QPT)], idx_v)
    sems = (sem0, sem1)

    def gather(q, b):
        pltpu.async_copy(tab_hbm.at[idx_v.at[pl.ds(q * GCHUNK, GCHUNK)]],
                         rows_v.at[b], sems[b])

    def drain(q, b):
        pltpu.make_async_copy(tab_hbm.at[idx_v.at[pl.ds(q * GCHUNK, GCHUNK)]],
                              rows_v.at[b], sems[b]).wait()
        pltpu.sync_copy(rows_v.at[b],
                        out_hbm.at[pl.ds(base0 + q * GCHUNK, GCHUNK)])

    gather(0, 0)

    def body(q, carry):
        for b in range(2):
            @pl.when(lax.rem(q, 2) == b)
            def _(b=b):
                @pl.when(q + 1 < QPT // GCHUNK)
                def _():
                    gather(q + 1, 1 - b)
                drain(q, b)
        return carry

    lax.fori_loop(0, QPT // GCHUNK, body, 0)


BR = 1000


def _scales(dg):
    d = dg[:, 0:1] + dg[:, 1:2] + 1.0
    dis = lax.rsqrt(d)
    return (1.0 - ALPHA) * dis, ALPHA * lax.sqrt(d) - (1.0 - ALPHA) * dis, dis


def _tc_stage1_body(x_ref, w_ref, b_ref, dg_ref, hs_ref):
    h = lax.dot_general(x_ref[...], w_ref[...], (((1,), (1,)), ((), ())),
                        preferred_element_type=jnp.float32) + b_ref[...]
    d = dg_ref[:, 0:1] + dg_ref[:, 1:2] + 1.0
    hs_ref[...] = h * lax.rsqrt(d)


def _tc_stage1(x, w1, b1r, degcols):
    return pl.pallas_call(
        _tc_stage1_body,
        grid=(N // BR,),
        in_specs=[
            pl.BlockSpec((BR, D), lambda i: (i, 0)),
            pl.BlockSpec((D, D), lambda i: (0, 0)),
            pl.BlockSpec((1, D), lambda i: (0, 0)),
            pl.BlockSpec((BR, 2), lambda i: (i, 0)),
        ],
        out_specs=pl.BlockSpec((BR, D), lambda i: (i, 0)),
        out_shape=jax.ShapeDtypeStruct((N, D), jnp.float32),
    )(x, w1, b1r, degcols)


def _tc_stage2_body(hs_ref, s0_ref, s1_ref, w_ref, b_ref, dg_ref, h2s_ref):
    ca, cb, dis = _scales(dg_ref[...])
    out1 = jnp.maximum(ca * (s0_ref[...] + s1_ref[...]) + cb * hs_ref[...],
                       0.0)
    h2 = lax.dot_general(out1, w_ref[...], (((1,), (1,)), ((), ())),
                         preferred_element_type=jnp.float32) + b_ref[...]
    h2s_ref[...] = h2 * dis


def _tc_stage2(h1s, s2n, w2, b2r, degcols):
    return pl.pallas_call(
        _tc_stage2_body,
        grid=(N // BR,),
        in_specs=[
            pl.BlockSpec((BR, D), lambda i: (i, 0)),
            pl.BlockSpec((BR, D), lambda i: (i, 0)),
            pl.BlockSpec((BR, D), lambda i: (i + N // BR, 0)),
            pl.BlockSpec((D, D), lambda i: (0, 0)),
            pl.BlockSpec((1, D), lambda i: (0, 0)),
            pl.BlockSpec((BR, 2), lambda i: (i, 0)),
        ],
        out_specs=pl.BlockSpec((BR, D), lambda i: (i, 0)),
        out_shape=jax.ShapeDtypeStruct((N, D), jnp.float32),
    )(h1s, s2n, s2n, w2, b2r, degcols)


def _tc_stage3_body(hs_ref, s0_ref, s1_ref, dg_ref, o_ref):
    ca, cb, _ = _scales(dg_ref[...])
    o_ref[...] = jnp.maximum(
        ca * (s0_ref[...] + s1_ref[...]) + cb * hs_ref[...], 0.0)


def _tc_stage3(h2s, s2n, degcols):
    return pl.pallas_call(
        _tc_stage3_body,
        grid=(N // BR,),
        in_specs=[
            pl.BlockSpec((BR, D), lambda i: (i, 0)),
            pl.BlockSpec((BR, D), lambda i: (i, 0)),
            pl.BlockSpec((BR, D), lambda i: (i + N // BR, 0)),
            pl.BlockSpec((BR, 2), lambda i: (i, 0)),
        ],
        out_specs=pl.BlockSpec((BR, D), lambda i: (i, 0)),
        out_shape=jax.ShapeDtypeStruct((N, D), jnp.float32),
    )(h2s, s2n, s2n, degcols)


def _tc_head_body(g0_ref, g1_ref, wa_ref, wb_ref, b3_ref, o_ref):
    logits = (
        lax.dot_general(g0_ref[...], wa_ref[...], (((1,), (1,)), ((), ())),
                        preferred_element_type=jnp.float32)
        + lax.dot_general(g1_ref[...], wb_ref[...], (((1,), (1,)), ((), ())),
                          preferred_element_type=jnp.float32)
        + b3_ref[...])
    m = jnp.max(logits, axis=1, keepdims=True)
    lse = m + jnp.log(jnp.sum(jnp.exp(logits - m), axis=1, keepdims=True))
    o_ref[...] = logits - lse


BQ = 2048


def _tc_head(g, w3a, w3b, b3r):
    return pl.pallas_call(
        _tc_head_body,
        grid=(Q // BQ,),
        in_specs=[
            pl.BlockSpec((BQ, D), lambda i: (i, 0)),
            pl.BlockSpec((BQ, D), lambda i: (i + Q // BQ, 0)),
            pl.BlockSpec((2, D), lambda i: (0, 0)),
            pl.BlockSpec((2, D), lambda i: (0, 0)),
            pl.BlockSpec((1, 2), lambda i: (0, 0)),
        ],
        out_specs=pl.BlockSpec((BQ, 2), lambda i: (i, 0)),
        out_shape=jax.ShapeDtypeStruct((Q, 2), jnp.float32),
    )(g, g, w3a, w3b, b3r)


def kernel(x, edge_index, index, W1, b1, W2, b2, W3, b3):
    ei = edge_index.astype(jnp.int32)
    ei1 = ei.reshape(2 * E)
    ei3 = ei1.reshape(2 * NW, NQ, CHUNK)
    idx_all = index.astype(jnp.int32).T.reshape(2 * Q)
    b1r = b1.reshape(1, D)
    b2r = b2.reshape(1, D)
    b3r = b3.reshape(1, 2)
    w3a = W3[:, :D]
    w3b = W3[:, D:]

    deg_flat = _sc_degree(ei3)
    degcols = deg_flat.reshape(NC, N_PAD).T[:N]

    h1s = _tc_stage1(x, W1, b1r, degcols)
    s = _sc_scatter(h1s, ei1)
    h2s = _tc_stage2(h1s, s, W2, b2r, degcols)
    s2 = _sc_scatter(h2s, ei1)
    out2 = _tc_stage3(h2s, s2, degcols)
    g = _sc_link(out2, idx_all)
    return _tc_head(g, w3a, w3b, b3r)
